# Initial kernel scaffold; baseline (speedup 1.0000x reference)
import jax
import jax.numpy as jnp
from jax.experimental import pallas as pl


def kernel(emissions, transitions, head_transitions, tail_transitions, cu_seqlens):
    raise NotImplementedError("write your pallas kernel here")


# speedup vs baseline: 97.4456x; 97.4456x over previous
import jax
import jax.numpy as jnp
from jax import lax
from jax.experimental import pallas as pl
from jax.experimental.pallas import tpu as pltpu

NT = 64
NB = 16
TOT = 32768
CHUNK = 512
LN2 = 0.6931471805599453


def _crf_body(cu_ref, lens_ref, em_hbm, trans_ref, head_ref, tail_ref,
              out_ref, embuf, sem):
    E = jnp.exp(trans_ref[...])
    eh = jnp.exp(head_ref[...])
    et = jnp.exp(tail_ref[...])
    lens = lens_ref[...]

    def mx(b, m):
        return jnp.maximum(m, cu_ref[b + 1] - cu_ref[b])
    maxlen = lax.fori_loop(0, NB, mx, jnp.int32(0))
    nch = lax.div(maxlen + (CHUNK - 1), CHUNK)

    def issue(ci, buf):
        for b in range(NB):
            off = jnp.minimum(cu_ref[b] + ci * CHUNK, TOT - CHUNK)
            pltpu.make_async_copy(
                em_hbm.at[pl.ds(off, CHUNK), :],
                embuf.at[buf, :, b, :],
                sem.at[buf],
            ).start()

    def wait(buf):
        for b in range(NB):
            pltpu.make_async_copy(
                em_hbm.at[pl.ds(0, CHUNK), :],
                embuf.at[buf, :, b, :],
                sem.at[buf],
            ).wait()

    issue(0, 0)

    def chunk_body(ci, carry):
        p, cf = carry
        buf = lax.rem(ci, 2)

        @pl.when(ci + 1 < nch)
        def _():
            issue(ci + 1, 1 - buf)

        wait(buf)
        t_hi = jnp.minimum(maxlen - ci * CHUNK, CHUNK)

        def step(t, c2):
            p, cf = c2
            tg = ci * CHUNK + t
            eem = jnp.exp(embuf[buf, t])
            q0 = lax.dot_general(p, E, (((1,), (0,)), ((), ())),
                                 preferred_element_type=jnp.float32,
                                 precision=lax.Precision.HIGHEST)
            q = jnp.where(tg == 0, eh, q0) * eem
            m = jnp.max(q, axis=1, keepdims=True)
            bits = lax.bitcast_convert_type(m, jnp.int32)
            ef = lax.shift_right_logical(bits, 23) & 0xFF
            scale = lax.bitcast_convert_type(
                lax.shift_left(254 - ef, 23), jnp.float32)
            act = tg < lens
            p = jnp.where(act, q * scale, p)
            cf = jnp.where(act, cf + (ef - 127).astype(jnp.float32), cf)
            return (p, cf)

        return lax.fori_loop(0, t_hi, step, (p, cf))

    p0 = jnp.zeros((NB, NT), jnp.float32)
    cf0 = jnp.zeros((NB, 1), jnp.float32)
    p, cf = lax.fori_loop(0, nch, chunk_body, (p0, cf0))
    s = jnp.sum(p * et, axis=1, keepdims=True)
    out_ref[...] = jnp.log(s) + cf * LN2


def kernel(emissions, transitions, head_transitions, tail_transitions,
           cu_seqlens):
    em = emissions.reshape(TOT, NT)
    trans = transitions.reshape(NT, NT)
    head = head_transitions.reshape(1, NT)
    tail = tail_transitions.reshape(1, NT)
    cu = cu_seqlens.astype(jnp.int32)
    lens = (cu[1:] - cu[:-1]).reshape(NB, 1)
    return pl.pallas_call(
        _crf_body,
        out_shape=jax.ShapeDtypeStruct((NB, 1), jnp.float32),
        in_specs=[
            pl.BlockSpec(memory_space=pltpu.SMEM),
            pl.BlockSpec(memory_space=pltpu.VMEM),
            pl.BlockSpec(memory_space=pltpu.MemorySpace.HBM),
            pl.BlockSpec(memory_space=pltpu.VMEM),
            pl.BlockSpec(memory_space=pltpu.VMEM),
            pl.BlockSpec(memory_space=pltpu.VMEM),
        ],
        out_specs=pl.BlockSpec(memory_space=pltpu.VMEM),
        scratch_shapes=[
            pltpu.VMEM((2, CHUNK, NB, NT), jnp.float32),
            pltpu.SemaphoreType.DMA((2,)),
        ],
    )(cu, lens, em, trans, head, tail)


# speedup vs baseline: 106.0648x; 1.0885x over previous
import jax
import jax.numpy as jnp
from jax import lax
from jax.experimental import pallas as pl
from jax.experimental.pallas import tpu as pltpu

NT = 64
NB = 16
TOT = 32768
CHUNK = 512
LN2 = 0.6931471805599453


def _crf_body(cu_ref, lens_ref, em_hbm, trans_ref, head_ref, tail_ref,
              out_ref, embuf, sem):
    E = jnp.exp(trans_ref[...])
    eh = jnp.exp(head_ref[...])
    et = jnp.exp(tail_ref[...])
    lens = lens_ref[...]

    def mx(b, m):
        return jnp.maximum(m, cu_ref[b + 1] - cu_ref[b])
    maxlen = lax.fori_loop(0, NB, mx, jnp.int32(0))
    nch = lax.div(maxlen + (CHUNK - 1), CHUNK)

    def issue(ci, buf):
        for b in range(NB):
            off = jnp.minimum(cu_ref[b] + ci * CHUNK, TOT - CHUNK)
            pltpu.make_async_copy(
                em_hbm.at[pl.ds(off, CHUNK), :],
                embuf.at[buf, :, b, :],
                sem.at[buf],
            ).start()

    def wait(buf):
        for b in range(NB):
            pltpu.make_async_copy(
                em_hbm.at[pl.ds(0, CHUNK), :],
                embuf.at[buf, :, b, :],
                sem.at[buf],
            ).wait()

    issue(0, 0)

    def chunk_body(ci, carry):
        p, cf = carry
        buf = lax.rem(ci, 2)

        @pl.when(ci + 1 < nch)
        def _():
            issue(ci + 1, 1 - buf)

        wait(buf)
        t_hi = jnp.minimum(maxlen - ci * CHUNK, CHUNK)

        def step(t, c2):
            p, cf = c2
            tg = ci * CHUNK + t
            eem = jnp.exp(embuf[buf, t])
            q0 = lax.dot_general(p, E, (((1,), (0,)), ((), ())),
                                 preferred_element_type=jnp.float32,
                                 precision=lax.Precision.DEFAULT)
            q = jnp.where(tg == 0, eh, q0) * eem
            m = jnp.max(q, axis=1, keepdims=True)
            bits = lax.bitcast_convert_type(m, jnp.int32)
            ef = lax.shift_right_logical(bits, 23) & 0xFF
            scale = lax.bitcast_convert_type(
                lax.shift_left(254 - ef, 23), jnp.float32)
            act = tg < lens
            p = jnp.where(act, q * scale, p)
            cf = jnp.where(act, cf + (ef - 127).astype(jnp.float32), cf)
            return (p, cf)

        return lax.fori_loop(0, t_hi, step, (p, cf))

    p0 = jnp.zeros((NB, NT), jnp.float32)
    cf0 = jnp.zeros((NB, 1), jnp.float32)
    p, cf = lax.fori_loop(0, nch, chunk_body, (p0, cf0))
    s = jnp.sum(p * et, axis=1, keepdims=True)
    out_ref[...] = jnp.log(s) + cf * LN2


def kernel(emissions, transitions, head_transitions, tail_transitions,
           cu_seqlens):
    em = emissions.reshape(TOT, NT)
    trans = transitions.reshape(NT, NT)
    head = head_transitions.reshape(1, NT)
    tail = tail_transitions.reshape(1, NT)
    cu = cu_seqlens.astype(jnp.int32)
    lens = (cu[1:] - cu[:-1]).reshape(NB, 1)
    return pl.pallas_call(
        _crf_body,
        out_shape=jax.ShapeDtypeStruct((NB, 1), jnp.float32),
        in_specs=[
            pl.BlockSpec(memory_space=pltpu.SMEM),
            pl.BlockSpec(memory_space=pltpu.VMEM),
            pl.BlockSpec(memory_space=pltpu.MemorySpace.HBM),
            pl.BlockSpec(memory_space=pltpu.VMEM),
            pl.BlockSpec(memory_space=pltpu.VMEM),
            pl.BlockSpec(memory_space=pltpu.VMEM),
        ],
        out_specs=pl.BlockSpec(memory_space=pltpu.VMEM),
        scratch_shapes=[
            pltpu.VMEM((2, CHUNK, NB, NT), jnp.float32),
            pltpu.SemaphoreType.DMA((2,)),
        ],
    )(cu, lens, em, trans, head, tail)


# speedup vs baseline: 152.0731x; 1.4338x over previous
import jax
import jax.numpy as jnp
from jax import lax
from jax.experimental import pallas as pl
from jax.experimental.pallas import tpu as pltpu

NT = 64
NB = 16
TOT = 32768
CHUNK = 512
GROUP = 4
LN2 = 0.6931471805599453


def _crf_body(cu_ref, lens_ref, em_hbm, trans_ref, head_ref, tail_ref,
              out_ref, embuf, sem):
    E = jnp.exp(trans_ref[...])
    eh = jnp.exp(head_ref[...])
    et = jnp.exp(tail_ref[...])
    lens = lens_ref[...]

    def mx(b, m):
        return jnp.maximum(m, cu_ref[b + 1] - cu_ref[b])
    maxlen = lax.fori_loop(0, NB, mx, jnp.int32(0))
    nch = lax.div(maxlen + (CHUNK - 1), CHUNK)

    def issue(ci, buf):
        for b in range(NB):
            off = jnp.minimum(cu_ref[b] + ci * CHUNK, TOT - CHUNK)
            pltpu.make_async_copy(
                em_hbm.at[pl.ds(off, CHUNK), :],
                embuf.at[buf, :, b, :],
                sem.at[buf],
            ).start()

    def wait(buf):
        for b in range(NB):
            pltpu.make_async_copy(
                em_hbm.at[pl.ds(0, CHUNK), :],
                embuf.at[buf, :, b, :],
                sem.at[buf],
            ).wait()

    issue(0, 0)

    def chunk_body(ci, carry):
        buf = lax.rem(ci, 2)

        @pl.when(ci + 1 < nch)
        def _():
            issue(ci + 1, 1 - buf)

        wait(buf)

        def group(g, c2):
            p, cf, psnap, cfsnap = c2
            q = p
            for k in range(GROUP):
                t = GROUP * g + k
                tg = ci * CHUNK + t
                eem = jnp.exp(embuf[buf, t])
                q0 = lax.dot_general(q, E, (((1,), (0,)), ((), ())),
                                     preferred_element_type=jnp.float32)
                q = jnp.where(tg == 0, eh, q0) * eem
                hit = tg == (lens - 1)
                psnap = jnp.where(hit, q, psnap)
                cfsnap = jnp.where(hit, cf, cfsnap)
            m = jnp.max(q, axis=1, keepdims=True)
            bits = lax.bitcast_convert_type(m, jnp.int32)
            ef = lax.shift_right_logical(bits, 23) & 0xFF
            scale = lax.bitcast_convert_type(
                lax.shift_left(254 - ef, 23), jnp.float32)
            p = q * scale
            cf = cf + (ef - 127).astype(jnp.float32)
            return (p, cf, psnap, cfsnap)

        return lax.fori_loop(0, CHUNK // GROUP, group, carry)

    init = (jnp.zeros((NB, NT), jnp.float32), jnp.zeros((NB, 1), jnp.float32),
            jnp.zeros((NB, NT), jnp.float32), jnp.zeros((NB, 1), jnp.float32))
    p, cf, psnap, cfsnap = lax.fori_loop(0, nch, chunk_body, init)
    s = jnp.sum(psnap * et, axis=1, keepdims=True)
    out_ref[...] = jnp.log(s) + cfsnap * LN2


def kernel(emissions, transitions, head_transitions, tail_transitions,
           cu_seqlens):
    em = emissions.reshape(TOT, NT)
    trans = transitions.reshape(NT, NT)
    head = head_transitions.reshape(1, NT)
    tail = tail_transitions.reshape(1, NT)
    cu = cu_seqlens.astype(jnp.int32)
    lens = (cu[1:] - cu[:-1]).reshape(NB, 1)
    return pl.pallas_call(
        _crf_body,
        out_shape=jax.ShapeDtypeStruct((NB, 1), jnp.float32),
        in_specs=[
            pl.BlockSpec(memory_space=pltpu.SMEM),
            pl.BlockSpec(memory_space=pltpu.VMEM),
            pl.BlockSpec(memory_space=pltpu.MemorySpace.HBM),
            pl.BlockSpec(memory_space=pltpu.VMEM),
            pl.BlockSpec(memory_space=pltpu.VMEM),
            pl.BlockSpec(memory_space=pltpu.VMEM),
        ],
        out_specs=pl.BlockSpec(memory_space=pltpu.VMEM),
        scratch_shapes=[
            pltpu.VMEM((2, CHUNK, NB, NT), jnp.float32),
            pltpu.SemaphoreType.DMA((2,)),
        ],
    )(cu, lens, em, trans, head, tail)


# speedup vs baseline: 162.9353x; 1.0714x over previous
import jax
import jax.numpy as jnp
from jax import lax
from jax.experimental import pallas as pl
from jax.experimental.pallas import tpu as pltpu

NT = 64
NB = 16
TOT = 32768
CHUNK = 512
GROUP = 8
LN2 = 0.6931471805599453


def _crf_body(cu_ref, lens_ref, em_hbm, trans_ref, head_ref, tail_ref,
              out_ref, embuf, sem):
    E = jnp.exp(trans_ref[...]).astype(jnp.bfloat16)
    eh = jnp.exp(head_ref[...])
    et = jnp.exp(tail_ref[...])
    lens = lens_ref[...]

    def mx(b, m):
        return jnp.maximum(m, cu_ref[b + 1] - cu_ref[b])
    maxlen = lax.fori_loop(0, NB, mx, jnp.int32(0))
    nch = lax.div(maxlen + (CHUNK - 1), CHUNK)

    def issue(ci, buf):
        for b in range(NB):
            off = jnp.minimum(cu_ref[b] + ci * CHUNK, TOT - CHUNK)
            pltpu.make_async_copy(
                em_hbm.at[pl.ds(off, CHUNK), :],
                embuf.at[buf, :, b, :],
                sem.at[buf],
            ).start()

    def wait(buf):
        for b in range(NB):
            pltpu.make_async_copy(
                em_hbm.at[pl.ds(0, CHUNK), :],
                embuf.at[buf, :, b, :],
                sem.at[buf],
            ).wait()

    issue(0, 0)

    def chunk_body(ci, carry):
        buf = lax.rem(ci, 2)

        @pl.when(ci + 1 < nch)
        def _():
            issue(ci + 1, 1 - buf)

        wait(buf)

        def group(g, c2):
            p, cf, psnap, cfsnap = c2
            q = p
            for k in range(GROUP):
                t = GROUP * g + k
                tg = ci * CHUNK + t
                eem = jnp.exp(embuf[buf, t])
                q0 = lax.dot_general(q.astype(jnp.bfloat16), E,
                                     (((1,), (0,)), ((), ())),
                                     preferred_element_type=jnp.float32)
                q = jnp.where(tg == 0, eh, q0) * eem
                hit = tg == (lens - 1)
                psnap = jnp.where(hit, q, psnap)
                cfsnap = jnp.where(hit, cf, cfsnap)
            m = jnp.max(q, axis=1, keepdims=True)
            bits = lax.bitcast_convert_type(m, jnp.int32)
            ef = lax.shift_right_logical(bits, 23) & 0xFF
            scale = lax.bitcast_convert_type(
                lax.shift_left(254 - ef, 23), jnp.float32)
            p = q * scale
            cf = cf + (ef - 127).astype(jnp.float32)
            return (p, cf, psnap, cfsnap)

        return lax.fori_loop(0, CHUNK // GROUP, group, carry)

    init = (jnp.zeros((NB, NT), jnp.float32), jnp.zeros((NB, 1), jnp.float32),
            jnp.zeros((NB, NT), jnp.float32), jnp.zeros((NB, 1), jnp.float32))
    p, cf, psnap, cfsnap = lax.fori_loop(0, nch, chunk_body, init)
    s = jnp.sum(psnap * et, axis=1, keepdims=True)
    out_ref[...] = jnp.log(s) + cfsnap * LN2


def kernel(emissions, transitions, head_transitions, tail_transitions,
           cu_seqlens):
    em = emissions.reshape(TOT, NT)
    trans = transitions.reshape(NT, NT)
    head = head_transitions.reshape(1, NT)
    tail = tail_transitions.reshape(1, NT)
    cu = cu_seqlens.astype(jnp.int32)
    lens = (cu[1:] - cu[:-1]).reshape(NB, 1)
    return pl.pallas_call(
        _crf_body,
        out_shape=jax.ShapeDtypeStruct((NB, 1), jnp.float32),
        in_specs=[
            pl.BlockSpec(memory_space=pltpu.SMEM),
            pl.BlockSpec(memory_space=pltpu.VMEM),
            pl.BlockSpec(memory_space=pltpu.MemorySpace.HBM),
            pl.BlockSpec(memory_space=pltpu.VMEM),
            pl.BlockSpec(memory_space=pltpu.VMEM),
            pl.BlockSpec(memory_space=pltpu.VMEM),
        ],
        out_specs=pl.BlockSpec(memory_space=pltpu.VMEM),
        scratch_shapes=[
            pltpu.VMEM((2, CHUNK, NB, NT), jnp.float32),
            pltpu.SemaphoreType.DMA((2,)),
        ],
    )(cu, lens, em, trans, head, tail)
